# Initial kernel scaffold; baseline (speedup 1.0000x reference)
import jax
import jax.numpy as jnp
from jax.experimental import pallas as pl


def kernel(x_author, x_institution, x_paper, src_affil, dst_affil, src_cites, dst_cites, src_rev_affil, dst_rev_affil, src_rev_writes, dst_rev_writes, src_writes, dst_writes, basis1, coeff1, bias1, basis2, coeff2, bias2):
    raise NotImplementedError("write your pallas kernel here")


# speedup vs baseline: 1.7530x; 1.7530x over previous
import jax
import jax.numpy as jnp
from jax import lax
from jax.experimental import pallas as pl
from jax.experimental.pallas import tpu as pltpu
from jax.experimental.pallas import tpu_sc as plsc

_NA, _NI, _NP = 10000, 1000, 10000
_PA, _PI, _PP = 10240, 1280, 10240
_E = 60000
_NSUB, _NCHUNK, _CH = 16, 30, 128
_EP = _NSUB * _NCHUNK * _CH

_REL_SRC = (0, 2, 1, 2, 0)
_REL_NPAD = (_PI, _PP, _PA, _PA, _PP)
_REL_NDST = (_NI, _NP, _NA, _NA, _NP)


def _make_sc_layer():
    mesh = plsc.VectorSubcoreMesh(core_axis_name="c", subcore_axis_name="s")
    out_type = [jax.ShapeDtypeStruct((npad, 2, 128), jnp.float32)
                for npad in _REL_NPAD]
    npair = _NCHUNK // 2
    scratch = [
        pltpu.VMEM_SHARED((_PA, 128), jnp.float32),
        pltpu.VMEM((npair, 2, _CH), jnp.int32),
        pltpu.VMEM((npair, 2, _CH), jnp.int32),
        pltpu.VMEM((_CH, 128), jnp.float32),
        pltpu.VMEM((_CH, 128), jnp.float32),
        pltpu.SemaphoreType.DMA,
        pltpu.SemaphoreType.DMA,
        pltpu.SemaphoreType.DMA,
        pltpu.SemaphoreType.DMA,
    ]

    def body(xa, xi, xp, sidx_h, didx_h, zeros_h, *rest):
        aggs = rest[0:5]
        (acc, sidx_v, didx_v, rows0, rows1,
         sem_g0, sem_g1, sem_s0, sem_s1) = rest[5:]
        tabs = (xa, xi, xp)
        c = lax.axis_index("c")
        s = lax.axis_index("s")
        for r in range(5):
            npad = _REL_NPAD[r]
            rows_per = npad // _NSUB
            base = s * rows_per
            pltpu.sync_copy(zeros_h.at[pl.ds(0, rows_per)],
                            acc.at[pl.ds(base, rows_per)])
            plsc.subcore_barrier()
            pltpu.sync_copy(sidx_h.at[r, c, s], sidx_v)
            pltpu.sync_copy(didx_h.at[r, s], didx_v)
            tab = tabs[_REL_SRC[r]]

            pltpu.async_copy(tab.at[sidx_v.at[0, 0]], rows0, sem_g0)
            for i in range(npair):
                if i > 0:
                    pltpu.make_async_copy(rows1, acc.at[didx_v.at[i, 1]],
                                          sem_s1).wait()
                pltpu.async_copy(tab.at[sidx_v.at[i, 1]], rows1, sem_g1)
                pltpu.make_async_copy(tab.at[sidx_v.at[i, 0]], rows0,
                                      sem_g0).wait()
                pltpu.async_copy(rows0, acc.at[didx_v.at[i, 0]], sem_s0,
                                 add=True)
                if i < npair - 1:
                    pltpu.make_async_copy(rows0, acc.at[didx_v.at[i, 0]],
                                          sem_s0).wait()
                    pltpu.async_copy(tab.at[sidx_v.at[i + 1, 0]], rows0,
                                     sem_g0)
                pltpu.make_async_copy(tab.at[sidx_v.at[i, 1]], rows1,
                                      sem_g1).wait()
                pltpu.async_copy(rows1, acc.at[didx_v.at[i, 1]], sem_s1,
                                 add=True)
            pltpu.make_async_copy(rows0, acc.at[didx_v.at[npair - 1, 0]],
                                  sem_s0).wait()
            pltpu.make_async_copy(rows1, acc.at[didx_v.at[npair - 1, 1]],
                                  sem_s1).wait()
            plsc.subcore_barrier()
            pltpu.sync_copy(acc.at[pl.ds(base, rows_per)],
                            aggs[r].at[pl.ds(base, rows_per), c])
            plsc.subcore_barrier()

    return pl.kernel(body, out_type=out_type, mesh=mesh, scratch_types=scratch)


def _make_deg():
    mesh = plsc.VectorSubcoreMesh(core_axis_name="c", subcore_axis_name="s")
    out_type = [jax.ShapeDtypeStruct((2, npad, 128), jnp.float32)
                for npad in _REL_NPAD]
    half = _NCHUNK // 2
    scratch = [
        pltpu.VMEM_SHARED((_PA, 128), jnp.float32),
        pltpu.VMEM((half, _CH), jnp.int32),
        pltpu.VMEM((_CH, 128), jnp.float32),
    ]

    def body(didx_h, zeros_h, ones_h, *rest):
        degs = rest[0:5]
        dacc, didx_v, ones_v = rest[5:]
        c = lax.axis_index("c")
        s = lax.axis_index("s")
        pltpu.sync_copy(ones_h, ones_v)
        for r in range(5):
            npad = _REL_NPAD[r]
            rows_per = npad // _NSUB
            base = s * rows_per
            pltpu.sync_copy(zeros_h.at[pl.ds(0, rows_per)],
                            dacc.at[pl.ds(base, rows_per)])
            plsc.subcore_barrier()
            pltpu.sync_copy(didx_h.at[r, c, s], didx_v)

            def chunk(j, carry):
                pltpu.sync_copy(ones_v, dacc.at[didx_v.at[j]], add=True)
                return carry

            lax.fori_loop(0, half, chunk, 0)
            plsc.subcore_barrier()
            pltpu.sync_copy(dacc.at[pl.ds(base, rows_per)],
                            degs[r].at[c, pl.ds(base, rows_per)])
            plsc.subcore_barrier()

    return pl.kernel(body, out_type=out_type, mesh=mesh, scratch_types=scratch)


def _tc_layer(aggs, degs, rels, basis, coeff, bias):
    npad = aggs[0].shape[0]
    d_out = basis.shape[2]
    br = 1280
    n_r = len(rels)

    def body(*refs):
        ad = refs[:2 * n_r]
        basis_ref, coeff_ref, bias_ref, out_ref = refs[2 * n_r:]
        xs = []
        for k in range(n_r):
            a = ad[2 * k][...]
            d = ad[2 * k + 1][...]
            xs.append(a / jnp.maximum(d, 1.0))
        acc = None
        for b in range(2):
            t = None
            for r, x in zip(rels, xs):
                term = coeff_ref[r, b] * x
                t = term if t is None else t + term
            mm = jnp.dot(t, basis_ref[b], preferred_element_type=jnp.float32)
            acc = mm if acc is None else acc + mm
        out_ref[...] = acc + bias_ref[...]

    in_specs = []
    for _ in rels:
        in_specs.append(pl.BlockSpec((br, 256), lambda i: (i, 0)))
        in_specs.append(pl.BlockSpec((br, 1), lambda i: (i, 0)))
    in_specs.append(pl.BlockSpec((2, 256, d_out), lambda i: (0, 0, 0)))
    in_specs.append(pl.BlockSpec(memory_space=pltpu.SMEM))
    in_specs.append(pl.BlockSpec((1, d_out), lambda i: (0, 0)))
    args = [v for pair in zip(aggs, degs) for v in pair]
    args += [basis, coeff, bias.reshape(1, d_out)]
    return pl.pallas_call(
        body,
        grid=(npad // br,),
        in_specs=in_specs,
        out_specs=pl.BlockSpec((br, d_out), lambda i: (i, 0)),
        out_shape=jax.ShapeDtypeStruct((npad, d_out), jnp.float32),
    )(*args)


@jax.jit
def _run(x_author, x_institution, x_paper, srcs, dsts,
         basis1, coeff1, bias1, basis2, coeff2, bias2):
    npair = _NCHUNK // 2
    sidx_list, didx_list = [], []
    for r in range(5):
        sp = jnp.concatenate(
            [srcs[r].astype(jnp.int32), jnp.zeros((_EP - _E,), jnp.int32)]) * 2
        sidx_list.append(
            jnp.stack([sp, sp + 1]).reshape(2, _NSUB, npair, 2, _CH))
        dp = jnp.concatenate(
            [dsts[r].astype(jnp.int32),
             jnp.full((_EP - _E,), _REL_NDST[r], jnp.int32)])
        didx_list.append(dp.reshape(_NSUB, npair, 2, _CH))
    sidx = jnp.stack(sidx_list)
    didx = jnp.stack(didx_list)
    zeros_h = jnp.zeros((_PA // _NSUB, 128), jnp.float32)
    ones_h = jnp.ones((_CH, 128), jnp.float32)

    didx_deg = didx.transpose(0, 3, 1, 2, 4)
    degs_raw = _make_deg()(didx_deg, zeros_h, ones_h)
    degs = [d[0, :, 0:1] + d[1, :, 0:1] for d in degs_raw]
    sc1 = _make_sc_layer()
    outs = sc1(x_author.reshape(2 * _NA, 128),
               x_institution.reshape(2 * _NI, 128),
               x_paper.reshape(2 * _NP, 128),
               sidx, didx, zeros_h)

    def dense(sc_outs, basis, coeff, bias):
        aggs = [o.reshape(npad, 256) for o, npad in zip(sc_outs, _REL_NPAD)]
        h_i = _tc_layer([aggs[0]], [degs[0]], [0], basis, coeff, bias)
        h_p = _tc_layer([aggs[1], aggs[4]], [degs[1], degs[4]], [1, 4],
                        basis, coeff, bias)
        h_a = _tc_layer([aggs[2], aggs[3]], [degs[2], degs[3]], [2, 3],
                        basis, coeff, bias)
        return h_a, h_i, h_p

    h_a, h_i, h_p = dense(outs[:5], basis1, coeff1, bias1)

    sc2 = _make_sc_layer()
    outs2 = sc2(h_a.reshape(2 * _PA, 128),
                h_i.reshape(2 * _PI, 128),
                h_p.reshape(2 * _PP, 128),
                sidx, didx, zeros_h)
    o_a, o_i, o_p = dense(outs2[:5], basis2, coeff2, bias2)
    return (o_a[:_NA], o_i[:_NI], o_p[:_NP])


def kernel(x_author, x_institution, x_paper, src_affil, dst_affil,
           src_cites, dst_cites, src_rev_affil, dst_rev_affil,
           src_rev_writes, dst_rev_writes, src_writes, dst_writes,
           basis1, coeff1, bias1, basis2, coeff2, bias2):
    srcs = (src_affil, src_cites, src_rev_affil, src_rev_writes, src_writes)
    dsts = (dst_affil, dst_cites, dst_rev_affil, dst_rev_writes, dst_writes)
    return _run(x_author, x_institution, x_paper, srcs, dsts,
                basis1, coeff1, bias1, basis2, coeff2, bias2)
